# baseline (speedup 1.0000x reference)
import functools

import jax
import jax.numpy as jnp
from jax import lax
from jax.experimental import pallas as pl
from jax.experimental.pallas import tpu as pltpu
from jax.experimental.pallas import tpu_sc as plsc

_PCODE = 2380
_D = 64
_B, _H, _W = 1024, 19, 19
_HW = _H * _W
_HWP = 368
_NV = _HWP // 16
_NC, _NS = 2, 16
_NW = _NC * _NS
_SPT = _B // _NW
_CHUNKS = ((0, 128), (128, 128), (256, 112))


def _body(sf0, sf1, bd0, bd1, table, out,
          sf0_v, sf1_v, bd0_v, bd1_v, idx0_v, idx1_v,
          rows0_v, rows1_v, outb_v, sem):
    wid = lax.axis_index("s") * _NC + lax.axis_index("c")
    lanes = lax.iota(jnp.int32, 16)
    bases = [(lanes + 16 * k) * _HW for k in range(4)]

    @pl.loop(0, _SPT)
    def _sample(s):
        b = wid * _SPT + s
        pltpu.sync_copy(sf0.at[b], sf0_v)
        pltpu.sync_copy(sf1.at[b], sf1_v)
        pltpu.sync_copy(bd0.at[b], bd0_v)
        pltpu.sync_copy(bd1.at[b], bd1_v)
        for i in range(_NV):
            sl = pl.ds(16 * i, 16)
            idx0_v[sl] = jnp.where(bd0_v[sl] > 0.0, _PCODE, sf0_v[sl])
            idx1_v[sl] = jnp.where(bd1_v[sl] > 0.0, _PCODE + (_PCODE + 1),
                                   sf1_v[sl] + (_PCODE + 1))
        copies = []
        for off, n in _CHUNKS:
            copies.append(pltpu.async_copy(
                table.at[idx0_v.at[pl.ds(off, n)]],
                rows0_v.at[pl.ds(off, n)], sem))
            copies.append(pltpu.async_copy(
                table.at[idx1_v.at[pl.ds(off, n)]],
                rows1_v.at[pl.ds(off, n)], sem))
        for c in copies:
            c.wait()

        @pl.loop(0, _HW)
        def _cell(n):
            for k in range(4):
                v = rows0_v[n, pl.ds(16 * k, 16)] + rows1_v[n, pl.ds(16 * k, 16)]
                plsc.store_scatter(outb_v, [bases[k] + n], v)

        pltpu.sync_copy(outb_v, out.at[b])


@jax.jit
def _pcode_embed(sf0, sf1, bd0, bd1, table):
    mesh = plsc.VectorSubcoreMesh(core_axis_name="c", subcore_axis_name="s",
                                  num_cores=_NC, num_subcores=_NS)
    f = pl.kernel(
        _body,
        out_type=jax.ShapeDtypeStruct((_B, _D * _HW), jnp.float32),
        mesh=mesh,
        compiler_params=pltpu.CompilerParams(needs_layout_passes=False,
                                             use_tc_tiling_on_sc=False),
        scratch_types=[
            pltpu.VMEM((_HWP,), jnp.int32),
            pltpu.VMEM((_HWP,), jnp.int32),
            pltpu.VMEM((_HWP,), jnp.float32),
            pltpu.VMEM((_HWP,), jnp.float32),
            pltpu.VMEM((_HWP,), jnp.int32),
            pltpu.VMEM((_HWP,), jnp.int32),
            pltpu.VMEM((_HWP, _D), jnp.float32),
            pltpu.VMEM((_HWP, _D), jnp.float32),
            pltpu.VMEM((_D * _HW,), jnp.float32),
            pltpu.SemaphoreType.DMA,
        ],
    )
    return f(sf0, sf1, bd0, bd1, table)


def kernel(sparse_feature_input, board_input, sparse_feature_dim, pcode_table):
    del sparse_feature_dim
    pad = ((0, 0), (0, _HWP - _HW))
    sf = sparse_feature_input.reshape(_B, 12, _HW)
    sf0 = jnp.pad(sf[:, 10], pad)
    sf1 = jnp.pad(sf[:, 11], pad)
    bd = board_input.reshape(_B, 2, _HW)
    bd0 = jnp.pad(bd[:, 0], pad)
    bd1 = jnp.pad(bd[:, 1], pad)
    out = _pcode_embed(sf0, sf1, bd0, bd1, pcode_table)
    return out.reshape(_B, _D, _H, _W)
